# baseline (speedup 1.0000x reference)
import functools

import jax
import jax.numpy as jnp
from jax.experimental import pallas as pl
from jax.experimental.pallas import tpu as pltpu
from jax.experimental.pallas import tpu_sc as plsc

NUM_NODE = 1024
TIME_DIM = 32
NUM_TIME = 288

NC = 2
NS = 16
LANES = 16

NW = NC * NS
REP = 128


def _sc_broadcast_lookup(time_flat, time_table, n_pairs):
    pairs_per_w = n_pairs // NW
    blocks_per_pair = NUM_NODE // REP

    mesh = plsc.VectorSubcoreMesh(
        core_axis_name="c", subcore_axis_name="s",
        num_cores=NC, num_subcores=NS,
    )

    @functools.partial(
        pl.kernel,
        out_type=jax.ShapeDtypeStruct((n_pairs, NUM_NODE, TIME_DIM),
                                      jnp.float32),
        mesh=mesh,
        compiler_params=pltpu.CompilerParams(use_tc_tiling_on_sc=False),
        scratch_types=[
            pltpu.VMEM((pairs_per_w,), jnp.int32),
            pltpu.VMEM((pairs_per_w, REP), jnp.int32),
            pltpu.VMEM((pairs_per_w * REP, TIME_DIM), jnp.float32),
            pltpu.SemaphoreType.DMA,
            pltpu.SemaphoreType.DMA,
        ],
    )
    def k(time_hbm, table_hbm, out_hbm, idx_v, idx_rep, rep, gsem, wsem):
        wid = jax.lax.axis_index("s") * NC + jax.lax.axis_index("c")
        base = wid * pairs_per_w

        pltpu.sync_copy(time_hbm.at[pl.ds(base, pairs_per_w)], idx_v)

        v0 = idx_v[pl.ds(0, LANES)]
        v1 = idx_v[pl.ds(pairs_per_w - LANES, LANES)]
        for i in range(pairs_per_w):
            x, li = (v0, i) if i < LANES else (v1, i - (pairs_per_w - LANES))
            scalar = jax.lax.squeeze(jax.lax.slice(x, (li,), (li + 1,)), (0,))
            splat = jax.lax.broadcast_in_dim(scalar, (LANES,), ())
            for j in range(REP // LANES):
                idx_rep[i, pl.ds(j * LANES, LANES)] = splat

        gathers = []
        for i in range(pairs_per_w):
            gathers.append(pltpu.async_copy(
                table_hbm.at[idx_rep.at[i]],
                rep.at[pl.ds(i * REP, REP)],
                gsem))
        for g in gathers:
            g.wait()

        writes = []
        for i in range(pairs_per_w):
            src = rep.at[pl.ds(i * REP, REP)]
            for j in range(blocks_per_pair):
                writes.append(pltpu.async_copy(
                    src,
                    out_hbm.at[base + i, pl.ds(j * REP, REP)],
                    wsem))
        for wdma in writes:
            wdma.wait()

    return k


def kernel(time, weekday, time_table):
    del weekday
    batch, seq, _ = time.shape
    n_pairs = batch * seq
    time_flat = time.reshape(n_pairs).astype(jnp.int32)
    out = _sc_broadcast_lookup(time_flat, time_table, n_pairs)(
        time_flat, time_table)
    return out.reshape(batch, seq, NUM_NODE, TIME_DIM)


# speedup vs baseline: 1.1740x; 1.1740x over previous
import functools

import jax
import jax.numpy as jnp
from jax.experimental import pallas as pl
from jax.experimental.pallas import tpu as pltpu
from jax.experimental.pallas import tpu_sc as plsc

NUM_NODE = 1024
TIME_DIM = 32

NC = 2
NS = 16
LANES = 16

NW = NC * NS
FILL = 256
NBUF = 2


def _sc_broadcast_lookup(time_flat, time_table, n_pairs):
    pairs_per_w = n_pairs // NW
    dmas_per_pair = NUM_NODE // FILL

    mesh = plsc.VectorSubcoreMesh(
        core_axis_name="c", subcore_axis_name="s",
        num_cores=NC, num_subcores=NS,
    )

    @functools.partial(
        pl.kernel,
        out_type=jax.ShapeDtypeStruct((n_pairs, NUM_NODE, TIME_DIM),
                                      jnp.float32),
        mesh=mesh,
        compiler_params=pltpu.CompilerParams(use_tc_tiling_on_sc=False),
        scratch_types=[
            pltpu.VMEM((pairs_per_w,), jnp.int32),
            pltpu.VMEM((pairs_per_w, TIME_DIM), jnp.float32),
            pltpu.VMEM((NBUF * FILL, TIME_DIM), jnp.float32),
            pltpu.SemaphoreType.DMA,
            pltpu.SemaphoreType.DMA,
            pltpu.SemaphoreType.DMA,
        ],
    )
    def k(time_hbm, table_hbm, out_hbm, idx_v, rows_v, rep, gsem, ws0, ws1):
        wid = jax.lax.axis_index("s") * NC + jax.lax.axis_index("c")
        base = wid * pairs_per_w

        pltpu.sync_copy(time_hbm.at[pl.ds(base, pairs_per_w)], idx_v)
        pltpu.async_copy(table_hbm.at[idx_v], rows_v, gsem).wait()

        wsems = [ws0, ws1]
        inflight = [None] * NBUF
        for i in range(pairs_per_w):
            b = i % NBUF
            if inflight[b] is not None:
                for d in inflight[b]:
                    d.wait()

            v_lo = rows_v[i, pl.ds(0, LANES)]
            v_hi = rows_v[i, pl.ds(LANES, LANES)]
            sect = b * FILL

            @pl.loop(0, FILL, unroll=8)
            def _fill(r):
                rep[sect + r, pl.ds(0, LANES)] = v_lo
                rep[sect + r, pl.ds(LANES, LANES)] = v_hi

            src = rep.at[pl.ds(sect, FILL)]
            inflight[b] = [
                pltpu.async_copy(
                    src, out_hbm.at[base + i, pl.ds(j * FILL, FILL)],
                    wsems[b])
                for j in range(dmas_per_pair)
            ]
        for dmas in inflight:
            if dmas is not None:
                for d in dmas:
                    d.wait()

    return k


def kernel(time, weekday, time_table):
    del weekday
    batch, seq, _ = time.shape
    n_pairs = batch * seq
    time_flat = time.reshape(n_pairs).astype(jnp.int32)
    out = _sc_broadcast_lookup(time_flat, time_table, n_pairs)(
        time_flat, time_table)
    return out.reshape(batch, seq, NUM_NODE, TIME_DIM)
